# Initial kernel scaffold; baseline (speedup 1.0000x reference)
import jax
import jax.numpy as jnp
from jax.experimental import pallas as pl


def kernel(wpts, viewdir, dists, part_pts, part_pbw, W1, b1, W2, b2, W3, b3, Wout, bout):
    raise NotImplementedError("write your pallas kernel here")


# speedup vs baseline: 23.7536x; 23.7536x over previous
import functools

import jax
import jax.numpy as jnp
from jax.experimental import pallas as pl
from jax.experimental.pallas import tpu as pltpu

_N = 8192
_P = 6
_M = 1024
_K = 4
_H = 128
_THRESH = 0.08
_BN = 512
_BIGF = 3.0e38
_BIGI = 2 ** 30


def _fused_body(feat_ref, dists_ref, pts_ref, nrm_ref,
                W1_ref, b1_ref, W2_ref, b2_ref, W3_ref, b3_ref,
                Wout_ref, bout_ref,
                raw_ref, occ_ref, occs_ref):
    feat = feat_ref[...]
    dists = dists_ref[...]
    iota = jax.lax.broadcasted_iota(jnp.int32, (1, _M), 1)

    best_occ = None
    best_raw = None
    for p in range(_P):
        d2 = None
        for c in range(3):
            xc = feat[:, c:c + 1]
            yc = pts_ref[p, c:c + 1, :]
            diff = xc - yc
            sq = diff * diff
            d2 = sq if d2 is None else d2 + sq

        nrmv = nrm_ref[p]

        wsum = None
        pacc = None
        for _ in range(_K):
            rmin = jnp.min(d2, axis=1, keepdims=True)
            sel = jnp.where(d2 == rmin, iota, _BIGI)
            imin = jnp.min(sel, axis=1, keepdims=True)
            hit = sel == imin
            nrm_k = jnp.sum(jnp.where(hit, nrmv, 0.0), axis=1,
                            keepdims=True)
            d2 = jnp.where(hit, _BIGF, d2)
            nd = jnp.sqrt(jnp.maximum(rmin, 1e-12))
            wk = 1.0 / (nd + 1e-8)
            wsum = wk if wsum is None else wsum + wk
            contrib = wk * nrm_k
            pacc = contrib if pacc is None else pacc + contrib
        pnorm = pacc / wsum
        tflag = (pnorm < _THRESH).astype(jnp.float32)

        dot = functools.partial(jax.lax.dot_general,
                                dimension_numbers=(((1,), (0,)), ((), ())),
                                preferred_element_type=jnp.float32)
        h = jax.nn.relu(dot(feat, W1_ref[p]) + b1_ref[p])
        h = jax.nn.relu(dot(h, W2_ref[p]) + b2_ref[p])
        h = jax.nn.relu(dot(h, W3_ref[p]) + b3_ref[p])
        out = dot(h, Wout_ref[p]) + bout_ref[p]

        rgb = jax.nn.sigmoid(out[:, 0:3])
        alpha = 1.0 - jnp.exp(-jax.nn.relu(out[:, 3:4]) * dists)
        occ_p = jax.nn.sigmoid(out[:, 4:5]) * tflag
        raw_p = jnp.concatenate([rgb, alpha], axis=1) * tflag

        occs_ref[:, p:p + 1] = occ_p

        if p == 0:
            best_occ = occ_p
            best_raw = raw_p
        else:
            better = occ_p > best_occ
            best_occ = jnp.where(better, occ_p, best_occ)
            best_raw = jnp.where(better, raw_p, best_raw)

    occ_ref[...] = best_occ
    raw_ref[...] = best_raw


def kernel(wpts, viewdir, dists, part_pts, part_pbw,
           W1, b1, W2, b2, W3, b3, Wout, bout):
    n = wpts.shape[0]
    feat = jnp.concatenate(
        [wpts, viewdir, jnp.zeros((n, 2), jnp.float32)], axis=1)
    dists2 = dists[:, None]
    pts_t = jnp.transpose(part_pts, (0, 2, 1))
    nrm = part_pbw[:, :, 24][:, None, :]
    W1p = jnp.concatenate(
        [W1, jnp.zeros((_P, 2, _H), jnp.float32)], axis=1)
    Woutp = jnp.concatenate(
        [Wout, jnp.zeros((_P, _H, 3), jnp.float32)], axis=2)
    boutp = jnp.concatenate(
        [bout, jnp.zeros((_P, 3), jnp.float32)], axis=1)

    nblk = n // _BN
    full = lambda *shape: pl.BlockSpec(shape, lambda i: (0,) * len(shape))

    raw, occ, occs = pl.pallas_call(
        _fused_body,
        grid=(nblk,),
        in_specs=[
            pl.BlockSpec((_BN, 8), lambda i: (i, 0)),
            pl.BlockSpec((_BN, 1), lambda i: (i, 0)),
            full(_P, 3, _M),
            full(_P, 1, _M),
            full(_P, 8, _H),
            full(_P, 1, _H),
            full(_P, _H, _H),
            full(_P, 1, _H),
            full(_P, _H, _H),
            full(_P, 1, _H),
            full(_P, _H, 8),
            full(_P, 1, 8),
        ],
        out_specs=[
            pl.BlockSpec((_BN, 4), lambda i: (i, 0)),
            pl.BlockSpec((_BN, 1), lambda i: (i, 0)),
            pl.BlockSpec((_BN, _P), lambda i: (i, 0)),
        ],
        out_shape=[
            jax.ShapeDtypeStruct((n, 4), jnp.float32),
            jax.ShapeDtypeStruct((n, 1), jnp.float32),
            jax.ShapeDtypeStruct((n, _P), jnp.float32),
        ],
        compiler_params=pltpu.CompilerParams(
            dimension_semantics=("parallel",)),
    )(feat, dists2, pts_t, nrm,
      W1p, b1[:, None, :], W2, b2[:, None, :], W3, b3[:, None, :],
      Woutp, boutp[:, None, :])

    return (raw, occ, occs[:, :, None])


# speedup vs baseline: 25.0119x; 1.0530x over previous
import functools

import jax
import jax.numpy as jnp
from jax.experimental import pallas as pl
from jax.experimental.pallas import tpu as pltpu

_N = 8192
_P = 6
_M = 1024
_K = 4
_H = 128
_THRESH = 0.08
_BN = 512
_BIGF = 3.0e38
_BIGI = 2 ** 30


def _fused_body(feat_ref, xt_ref, dists_ref, pts_ref, nrm_ref,
                W1_ref, b1_ref, W2_ref, b2_ref, W3_ref, b3_ref,
                Wout_ref, bout_ref,
                raw_ref, occ_ref, occs_ref):
    feat = feat_ref[...]
    xt = xt_ref[...]
    dists = dists_ref[...]
    iota_s = jax.lax.broadcasted_iota(jnp.int32, (_M, 1), 0)

    pnorm_rows = []
    for p in range(_P):
        d2 = None
        for c in range(3):
            xc = xt[c:c + 1, :]
            yc = pts_ref[p, :, c:c + 1]
            diff = xc - yc
            sq = diff * diff
            d2 = sq if d2 is None else d2 + sq

        nrmv = nrm_ref[p]

        wsum = None
        pacc = None
        for _ in range(_K):
            rmin = jnp.min(d2, axis=0, keepdims=True)
            sel = jnp.where(d2 == rmin, iota_s, _BIGI)
            imin = jnp.min(sel, axis=0, keepdims=True)
            hit = sel == imin
            nrm_k = jnp.sum(jnp.where(hit, nrmv, 0.0), axis=0,
                            keepdims=True)
            d2 = jnp.where(hit, _BIGF, d2)
            nd = jnp.sqrt(jnp.maximum(rmin, 1e-12))
            wk = 1.0 / (nd + 1e-8)
            wsum = wk if wsum is None else wsum + wk
            contrib = wk * nrm_k
            pacc = contrib if pacc is None else pacc + contrib
        pnorm_rows.append(pacc / wsum)

    pn = jnp.concatenate(
        pnorm_rows + [jnp.zeros((8 - _P, _BN), jnp.float32)], axis=0)
    tf = (pn < _THRESH).astype(jnp.float32)
    tft = tf.T

    dot = functools.partial(jax.lax.dot_general,
                            dimension_numbers=(((1,), (0,)), ((), ())),
                            preferred_element_type=jnp.float32)
    best_occ = None
    best_raw = None
    for p in range(_P):
        tflag = tft[:, p:p + 1]
        h = jax.nn.relu(dot(feat, W1_ref[p]) + b1_ref[p])
        h = jax.nn.relu(dot(h, W2_ref[p]) + b2_ref[p])
        h = jax.nn.relu(dot(h, W3_ref[p]) + b3_ref[p])
        out = dot(h, Wout_ref[p]) + bout_ref[p]

        rgb = jax.nn.sigmoid(out[:, 0:3])
        alpha = 1.0 - jnp.exp(-jax.nn.relu(out[:, 3:4]) * dists)
        occ_p = jax.nn.sigmoid(out[:, 4:5]) * tflag
        raw_p = jnp.concatenate([rgb, alpha], axis=1) * tflag

        occs_ref[:, p:p + 1] = occ_p

        if p == 0:
            best_occ = occ_p
            best_raw = raw_p
        else:
            better = occ_p > best_occ
            best_occ = jnp.where(better, occ_p, best_occ)
            best_raw = jnp.where(better, raw_p, best_raw)

    occ_ref[...] = best_occ
    raw_ref[...] = best_raw


def kernel(wpts, viewdir, dists, part_pts, part_pbw,
           W1, b1, W2, b2, W3, b3, Wout, bout):
    n = wpts.shape[0]
    feat = jnp.concatenate(
        [wpts, viewdir, jnp.zeros((n, 2), jnp.float32)], axis=1)
    xt = wpts.T
    dists2 = dists[:, None]
    nrm = part_pbw[:, :, 24:25]
    W1p = jnp.concatenate(
        [W1, jnp.zeros((_P, 2, _H), jnp.float32)], axis=1)
    Woutp = jnp.concatenate(
        [Wout, jnp.zeros((_P, _H, 3), jnp.float32)], axis=2)
    boutp = jnp.concatenate(
        [bout, jnp.zeros((_P, 3), jnp.float32)], axis=1)

    nblk = n // _BN
    full = lambda *shape: pl.BlockSpec(shape, lambda i: (0,) * len(shape))

    raw, occ, occs = pl.pallas_call(
        _fused_body,
        grid=(nblk,),
        in_specs=[
            pl.BlockSpec((_BN, 8), lambda i: (i, 0)),
            pl.BlockSpec((3, _BN), lambda i: (0, i)),
            pl.BlockSpec((_BN, 1), lambda i: (i, 0)),
            full(_P, _M, 3),
            full(_P, _M, 1),
            full(_P, 8, _H),
            full(_P, 1, _H),
            full(_P, _H, _H),
            full(_P, 1, _H),
            full(_P, _H, _H),
            full(_P, 1, _H),
            full(_P, _H, 8),
            full(_P, 1, 8),
        ],
        out_specs=[
            pl.BlockSpec((_BN, 4), lambda i: (i, 0)),
            pl.BlockSpec((_BN, 1), lambda i: (i, 0)),
            pl.BlockSpec((_BN, _P), lambda i: (i, 0)),
        ],
        out_shape=[
            jax.ShapeDtypeStruct((n, 4), jnp.float32),
            jax.ShapeDtypeStruct((n, 1), jnp.float32),
            jax.ShapeDtypeStruct((n, _P), jnp.float32),
        ],
        compiler_params=pltpu.CompilerParams(
            dimension_semantics=("parallel",)),
    )(feat, xt, dists2, part_pts, nrm,
      W1p, b1[:, None, :], W2, b2[:, None, :], W3, b3[:, None, :],
      Woutp, boutp[:, None, :])

    return (raw, occ, occs[:, :, None])


# speedup vs baseline: 41.1386x; 1.6448x over previous
import functools

import jax
import jax.numpy as jnp
from jax.experimental import pallas as pl
from jax.experimental.pallas import tpu as pltpu

_N = 8192
_P = 6
_M = 1024
_K = 4
_H = 128
_THRESH = 0.08
_BN = 512
_BIGF = 3.0e38
_BIGI = 2 ** 30


def _build_d2(xt, pts_ref, p):
    d2 = None
    for c in range(3):
        xc = xt[c:c + 1, :]
        yc = pts_ref[p, :, c:c + 1]
        diff = xc - yc
        sq = diff * diff
        d2 = sq if d2 is None else d2 + sq
    return d2


def _wk(rmin):
    nd = jnp.sqrt(jnp.maximum(rmin, 1e-12))
    return 1.0 / (nd + 1e-8)


def _fused_body(feat_ref, xt_ref, dists_ref, pts_ref, onrm_ref,
                W1_ref, b1_ref, W2_ref, b2_ref, W3_ref, b3_ref,
                Wout_ref, bout_ref,
                raw_ref, occ_ref, occs_ref, pn_ref):
    feat = feat_ref[...]
    xt = xt_ref[...]
    dists = dists_ref[...]

    dotg = functools.partial(jax.lax.dot_general,
                             dimension_numbers=(((1,), (0,)), ((), ())),
                             preferred_element_type=jnp.float32)

    all_ok = None
    for p in range(_P):
        d2 = _build_d2(xt, pts_ref, p)
        onrm = onrm_ref[p]
        wsum = None
        pacc = None
        ok = None
        for k in range(_K):
            rmin = jnp.min(d2, axis=0, keepdims=True)
            eq = d2 == rmin
            eqf = jnp.where(eq, 1.0, 0.0)
            cn = dotg(onrm, eqf)
            cnt = cn[0:1, :]
            nrm_k = cn[1:2, :]
            okk = jnp.all(cnt == 1.0)
            ok = okk if ok is None else jnp.logical_and(ok, okk)
            if k + 1 < _K:
                d2 = jnp.where(eq, _BIGF, d2)
            wk = _wk(rmin)
            wsum = wk if wsum is None else wsum + wk
            contrib = wk * nrm_k
            pacc = contrib if pacc is None else pacc + contrib
        pn_ref[p:p + 1, :] = pacc / wsum
        all_ok = ok if all_ok is None else jnp.logical_and(all_ok, ok)

    @pl.when(jnp.logical_not(all_ok))
    def _slow():
        iota_s = jax.lax.broadcasted_iota(jnp.int32, (_M, 1), 0)
        for p in range(_P):
            d2 = _build_d2(xt, pts_ref, p)
            nrmv = onrm_ref[p, 1:2, :].T
            wsum = None
            pacc = None
            for _ in range(_K):
                rmin = jnp.min(d2, axis=0, keepdims=True)
                sel = jnp.where(d2 == rmin, iota_s, _BIGI)
                imin = jnp.min(sel, axis=0, keepdims=True)
                hit = sel == imin
                nrm_k = jnp.sum(jnp.where(hit, nrmv, 0.0), axis=0,
                                keepdims=True)
                d2 = jnp.where(hit, _BIGF, d2)
                wk = _wk(rmin)
                wsum = wk if wsum is None else wsum + wk
                contrib = wk * nrm_k
                pacc = contrib if pacc is None else pacc + contrib
            pn_ref[p:p + 1, :] = pacc / wsum

    tf = (pn_ref[...] < _THRESH).astype(jnp.float32)
    tft = tf.T

    best_occ = None
    best_raw = None
    for p in range(_P):
        tflag = tft[:, p:p + 1]
        h = jax.nn.relu(dotg(feat, W1_ref[p]) + b1_ref[p])
        h = jax.nn.relu(dotg(h, W2_ref[p]) + b2_ref[p])
        h = jax.nn.relu(dotg(h, W3_ref[p]) + b3_ref[p])
        out = dotg(h, Wout_ref[p]) + bout_ref[p]

        rgb = jax.nn.sigmoid(out[:, 0:3])
        alpha = 1.0 - jnp.exp(-jax.nn.relu(out[:, 3:4]) * dists)
        occ_p = jax.nn.sigmoid(out[:, 4:5]) * tflag
        raw_p = jnp.concatenate([rgb, alpha], axis=1) * tflag

        occs_ref[:, p:p + 1] = occ_p

        if p == 0:
            best_occ = occ_p
            best_raw = raw_p
        else:
            better = occ_p > best_occ
            best_occ = jnp.where(better, occ_p, best_occ)
            best_raw = jnp.where(better, raw_p, best_raw)

    occ_ref[...] = best_occ
    raw_ref[...] = best_raw


def kernel(wpts, viewdir, dists, part_pts, part_pbw,
           W1, b1, W2, b2, W3, b3, Wout, bout):
    n = wpts.shape[0]
    feat = jnp.concatenate(
        [wpts, viewdir, jnp.zeros((n, 2), jnp.float32)], axis=1)
    xt = wpts.T
    dists2 = dists[:, None]
    onrm = jnp.concatenate(
        [jnp.ones((_P, 1, _M), jnp.float32),
         part_pbw[:, :, 24].reshape(_P, 1, _M)], axis=1)
    W1p = jnp.concatenate(
        [W1, jnp.zeros((_P, 2, _H), jnp.float32)], axis=1)
    Woutp = jnp.concatenate(
        [Wout, jnp.zeros((_P, _H, 3), jnp.float32)], axis=2)
    boutp = jnp.concatenate(
        [bout, jnp.zeros((_P, 3), jnp.float32)], axis=1)

    nblk = n // _BN
    full = lambda *shape: pl.BlockSpec(shape, lambda i: (0,) * len(shape))

    raw, occ, occs = pl.pallas_call(
        _fused_body,
        grid=(nblk,),
        in_specs=[
            pl.BlockSpec((_BN, 8), lambda i: (i, 0)),
            pl.BlockSpec((3, _BN), lambda i: (0, i)),
            pl.BlockSpec((_BN, 1), lambda i: (i, 0)),
            full(_P, _M, 3),
            full(_P, 2, _M),
            full(_P, 8, _H),
            full(_P, 1, _H),
            full(_P, _H, _H),
            full(_P, 1, _H),
            full(_P, _H, _H),
            full(_P, 1, _H),
            full(_P, _H, 8),
            full(_P, 1, 8),
        ],
        out_specs=[
            pl.BlockSpec((_BN, 4), lambda i: (i, 0)),
            pl.BlockSpec((_BN, 1), lambda i: (i, 0)),
            pl.BlockSpec((_BN, _P), lambda i: (i, 0)),
        ],
        out_shape=[
            jax.ShapeDtypeStruct((n, 4), jnp.float32),
            jax.ShapeDtypeStruct((n, 1), jnp.float32),
            jax.ShapeDtypeStruct((n, _P), jnp.float32),
        ],
        scratch_shapes=[pltpu.VMEM((8, _BN), jnp.float32)],
        compiler_params=pltpu.CompilerParams(
            dimension_semantics=("parallel",)),
    )(feat, xt, dists2, part_pts, onrm,
      W1p, b1[:, None, :], W2, b2[:, None, :], W3, b3[:, None, :],
      Woutp, boutp[:, None, :])

    return (raw, occ, occs[:, :, None])
